# baseline (speedup 1.0000x reference)
import jax
import jax.numpy as jnp
from jax.experimental import pallas as pl
from jax.experimental.pallas import tpu as pltpu

D = 768
FF = 3072
E = 16

F_BLK_A = 768
F_BLK_B = 3072


def _prologue_kernel(xf_ref, time_c_ref, ada_w_ref, ada_b_ref, gate_w_ref,
                     sw1_ref, sw3_ref, sw2_ref,
                     h_ref, routing_ref, xps_ref):
    step = pl.program_id(0)
    T = xf_ref.shape[0]
    B = time_c_ref.shape[0]
    L = T // B

    @pl.when(step == 0)
    def _():
        cond = jax.nn.silu(time_c_ref[:])
        ss = jax.lax.dot_general(cond, ada_w_ref[:], (((1,), (1,)), ((), ())),
                                 preferred_element_type=jnp.float32)
        ss = ss + ada_b_ref[:][None, :]
        shift = ss[:, :D]
        scale = ss[:, D:]
        rows = jax.lax.broadcasted_iota(jnp.int32, (T, B), 0) // L
        cols = jax.lax.broadcasted_iota(jnp.int32, (T, B), 1)
        P = (rows == cols).astype(jnp.float32)
        shift_t = jax.lax.dot_general(P, shift, (((1,), (0,)), ((), ())),
                                      preferred_element_type=jnp.float32)
        scale_t = jax.lax.dot_general(P, scale, (((1,), (0,)), ((), ())),
                                      preferred_element_type=jnp.float32)
        xf = xf_ref[:]
        m = jnp.mean(xf, axis=-1, keepdims=True)
        v = jnp.mean((xf - m) ** 2, axis=-1, keepdims=True)
        xln = (xf - m) * jax.lax.rsqrt(v + 1e-5)
        h = xln * (1.0 + scale_t) + shift_t
        h_ref[:] = h.astype(jnp.bfloat16)

        logits = jax.lax.dot_general(h, gate_w_ref[:], (((1,), (1,)), ((), ())),
                                     preferred_element_type=jnp.float32)
        ids = jax.lax.broadcasted_iota(jnp.int32, (T, E), 1)
        l1 = jnp.max(logits, axis=1, keepdims=True)
        i1 = jnp.min(jnp.where(logits == l1, ids, E), axis=1, keepdims=True)
        is1 = ids == i1
        masked = jnp.where(is1, jnp.float32(-1e30), logits)
        l2 = jnp.max(masked, axis=1, keepdims=True)
        i2 = jnp.min(jnp.where(masked == l2, ids, E), axis=1, keepdims=True)
        is2 = ids == i2
        wa = jax.nn.sigmoid(l1 - l2)
        routing_ref[:] = jnp.where(is1, wa, 0.0) + jnp.where(is2, 1.0 - wa, 0.0)

        xps_ref[:] = xf

    hb = h_ref[:]
    g = jax.lax.dot_general(hb, sw1_ref[:].astype(jnp.bfloat16),
                            (((1,), (1,)), ((), ())),
                            preferred_element_type=jnp.float32)
    u = jax.lax.dot_general(hb, sw3_ref[:].astype(jnp.bfloat16),
                            (((1,), (1,)), ((), ())),
                            preferred_element_type=jnp.float32)
    act = (jax.nn.silu(g) * u).astype(jnp.bfloat16)
    xps_ref[:] += jax.lax.dot_general(act, sw2_ref[:].astype(jnp.bfloat16),
                                      (((1,), (1,)), ((), ())),
                                      preferred_element_type=jnp.float32)


def _moe_kernel(h_ref, routing_ref, xps_ref, w1_ref, w3_ref, w2_ref, out_ref):
    e = pl.program_id(0)
    f = pl.program_id(1)
    T = h_ref.shape[0]

    @pl.when((e == 0) & (f == 0))
    def _():
        out_ref[:] = xps_ref[:]

    hb = h_ref[:]
    g = jax.lax.dot_general(hb, w1_ref[0].astype(jnp.bfloat16),
                            (((1,), (1,)), ((), ())),
                            preferred_element_type=jnp.float32)
    u = jax.lax.dot_general(hb, w3_ref[0].astype(jnp.bfloat16),
                            (((1,), (1,)), ((), ())),
                            preferred_element_type=jnp.float32)
    act = jax.nn.silu(g) * u
    lane = jax.lax.broadcasted_iota(jnp.int32, (T, E), 1)
    col = jnp.sum(jnp.where(lane == e, routing_ref[:], 0.0), axis=1,
                  keepdims=True)
    act = (act * col).astype(jnp.bfloat16)
    out_ref[:] += jax.lax.dot_general(act, w2_ref[0].astype(jnp.bfloat16),
                                      (((1,), (1,)), ((), ())),
                                      preferred_element_type=jnp.float32)


def kernel(x, time_c, ada_w, ada_b, gate_w, w1, w3, w2, sw1, sw3, sw2):
    B, L, Dm = x.shape
    T = B * L
    xf = x.reshape(T, Dm)

    nfa = FF // F_BLK_A
    h, routing, xps = pl.pallas_call(
        _prologue_kernel,
        grid=(nfa,),
        in_specs=[
            pl.BlockSpec((T, Dm), lambda f: (0, 0)),
            pl.BlockSpec(time_c.shape, lambda f: (0, 0)),
            pl.BlockSpec(ada_w.shape, lambda f: (0, 0)),
            pl.BlockSpec(ada_b.shape, lambda f: (0,)),
            pl.BlockSpec(gate_w.shape, lambda f: (0, 0)),
            pl.BlockSpec((F_BLK_A, Dm), lambda f: (f, 0)),
            pl.BlockSpec((F_BLK_A, Dm), lambda f: (f, 0)),
            pl.BlockSpec((Dm, F_BLK_A), lambda f: (0, f)),
        ],
        out_specs=[
            pl.BlockSpec((T, Dm), lambda f: (0, 0)),
            pl.BlockSpec((T, E), lambda f: (0, 0)),
            pl.BlockSpec((T, Dm), lambda f: (0, 0)),
        ],
        out_shape=[
            jax.ShapeDtypeStruct((T, Dm), jnp.bfloat16),
            jax.ShapeDtypeStruct((T, E), jnp.float32),
            jax.ShapeDtypeStruct((T, Dm), jnp.float32),
        ],
        compiler_params=pltpu.CompilerParams(
            dimension_semantics=("arbitrary",)),
    )(xf, time_c, ada_w, ada_b, gate_w, sw1, sw3, sw2)

    nfb = FF // F_BLK_B
    y = pl.pallas_call(
        _moe_kernel,
        grid=(E, nfb),
        in_specs=[
            pl.BlockSpec((T, Dm), lambda e, f: (0, 0)),
            pl.BlockSpec((T, E), lambda e, f: (0, 0)),
            pl.BlockSpec((T, Dm), lambda e, f: (0, 0)),
            pl.BlockSpec((1, F_BLK_B, Dm), lambda e, f: (e, f, 0)),
            pl.BlockSpec((1, F_BLK_B, Dm), lambda e, f: (e, f, 0)),
            pl.BlockSpec((1, Dm, F_BLK_B), lambda e, f: (e, 0, f)),
        ],
        out_specs=pl.BlockSpec((T, Dm), lambda e, f: (0, 0)),
        out_shape=jax.ShapeDtypeStruct((T, Dm), jnp.float32),
        compiler_params=pltpu.CompilerParams(
            dimension_semantics=("arbitrary", "arbitrary")),
    )(h, routing, xps, w1, w3, w2)

    return y.reshape(B, L, Dm)


# speedup vs baseline: 1.0496x; 1.0496x over previous
import jax
import jax.numpy as jnp
from jax.experimental import pallas as pl
from jax.experimental.pallas import tpu as pltpu

D = 768
FF = 3072
E = 16
HC = 1536
NHALF = FF // HC
NCHUNK = NHALF * E + NHALF
NBUF = 3


def _fused_kernel(xf_ref, time_c_ref, ada_w_ref, ada_b_ref, gate_w_ref,
                  w1h, w3h, w2h, swh1, swh3, swh2,
                  out_ref,
                  b1, b3, b2, h_ref, routing_ref, acc_ref, sems):
    T = xf_ref.shape[0]
    B = time_c_ref.shape[0]
    L = T // B

    def copy_descs(i, slot):
        e = jnp.minimum(i // NHALF, E - 1)
        half = (i % NHALF) * HC
        routed = [
            pltpu.make_async_copy(w1h.at[e, pl.ds(half, HC), :],
                                  b1.at[slot], sems.at[slot, 0]),
            pltpu.make_async_copy(w3h.at[e, pl.ds(half, HC), :],
                                  b3.at[slot], sems.at[slot, 1]),
            pltpu.make_async_copy(w2h.at[e, :, pl.ds(half, HC)],
                                  b2.at[slot], sems.at[slot, 2]),
        ]
        shared = [
            pltpu.make_async_copy(swh1.at[pl.ds(half, HC), :],
                                  b1.at[slot], sems.at[slot, 0]),
            pltpu.make_async_copy(swh3.at[pl.ds(half, HC), :],
                                  b3.at[slot], sems.at[slot, 1]),
            pltpu.make_async_copy(swh2.at[:, pl.ds(half, HC)],
                                  b2.at[slot], sems.at[slot, 2]),
        ]
        return routed, shared

    def start_chunk(i, slot):
        routed, shared = copy_descs(i, slot)

        @pl.when(i < NHALF * E)
        def _():
            for c in routed:
                c.start()

        @pl.when(i >= NHALF * E)
        def _():
            for c in shared:
                c.start()

    def wait_chunk(i, slot):
        routed, shared = copy_descs(i, slot)

        @pl.when(i < NHALF * E)
        def _():
            for c in routed:
                c.wait()

        @pl.when(i >= NHALF * E)
        def _():
            for c in shared:
                c.wait()

    cond = jax.nn.silu(time_c_ref[:])
    ss = jax.lax.dot_general(cond, ada_w_ref[:], (((1,), (1,)), ((), ())),
                             preferred_element_type=jnp.float32)
    ss = ss + ada_b_ref[:][None, :]
    shift = ss[:, :D]
    scale = ss[:, D:]
    rows = jax.lax.broadcasted_iota(jnp.int32, (T, B), 0) // L
    cols = jax.lax.broadcasted_iota(jnp.int32, (T, B), 1)
    P = (rows == cols).astype(jnp.float32)
    shift_t = jax.lax.dot_general(P, shift, (((1,), (0,)), ((), ())),
                                  preferred_element_type=jnp.float32)
    scale_t = jax.lax.dot_general(P, scale, (((1,), (0,)), ((), ())),
                                  preferred_element_type=jnp.float32)
    xf = xf_ref[:]
    m = jnp.mean(xf, axis=-1, keepdims=True)
    v = jnp.mean((xf - m) ** 2, axis=-1, keepdims=True)
    xln = (xf - m) * jax.lax.rsqrt(v + 1e-5)
    h = xln * (1.0 + scale_t) + shift_t
    h_ref[:] = h.astype(jnp.bfloat16)

    logits = jax.lax.dot_general(h, gate_w_ref[:], (((1,), (1,)), ((), ())),
                                 preferred_element_type=jnp.float32)
    ids = jax.lax.broadcasted_iota(jnp.int32, (T, E), 1)
    l1 = jnp.max(logits, axis=1, keepdims=True)
    i1 = jnp.min(jnp.where(logits == l1, ids, E), axis=1, keepdims=True)
    is1 = ids == i1
    masked = jnp.where(is1, jnp.float32(-1e30), logits)
    l2 = jnp.max(masked, axis=1, keepdims=True)
    i2 = jnp.min(jnp.where(masked == l2, ids, E), axis=1, keepdims=True)
    is2 = ids == i2
    wa = jax.nn.sigmoid(l1 - l2)
    routing_ref[:] = jnp.where(is1, wa, 0.0) + jnp.where(is2, 1.0 - wa, 0.0)

    acc_ref[:] = xf

    for i in range(NBUF - 1):
        start_chunk(jnp.int32(i), jnp.int32(i))

    def body(i, carry):
        slot = jax.lax.rem(i, NBUF)
        wait_chunk(i, slot)
        nxt = i + (NBUF - 1)

        @pl.when(nxt < NCHUNK)
        def _():
            start_chunk(nxt, jax.lax.rem(nxt, NBUF))

        hb = h_ref[:]
        g = jax.lax.dot_general(hb, b1[slot].astype(jnp.bfloat16),
                                (((1,), (1,)), ((), ())),
                                preferred_element_type=jnp.float32)
        u = jax.lax.dot_general(hb, b3[slot].astype(jnp.bfloat16),
                                (((1,), (1,)), ((), ())),
                                preferred_element_type=jnp.float32)
        act = (jax.nn.silu(g) * u).astype(jnp.bfloat16)
        partial = jax.lax.dot_general(act, b2[slot].astype(jnp.bfloat16),
                                      (((1,), (1,)), ((), ())),
                                      preferred_element_type=jnp.float32)
        e = i // NHALF
        lane = jax.lax.broadcasted_iota(jnp.int32, (T, E), 1)
        col = jnp.sum(jnp.where(lane == e, routing_ref[:], 0.0), axis=1,
                      keepdims=True)
        col = col + (i >= NHALF * E).astype(jnp.float32)
        acc_ref[:] += partial * col
        return carry

    jax.lax.fori_loop(0, NCHUNK, body, 0)
    out_ref[:] = acc_ref[:]


def kernel(x, time_c, ada_w, ada_b, gate_w, w1, w3, w2, sw1, sw3, sw2):
    B, L, Dm = x.shape
    T = B * L
    xf = x.reshape(T, Dm)

    vmem = pl.BlockSpec(memory_space=pltpu.MemorySpace.VMEM)
    hbm = pl.BlockSpec(memory_space=pltpu.MemorySpace.HBM)

    y = pl.pallas_call(
        _fused_kernel,
        in_specs=[vmem, vmem, vmem, vmem, vmem, hbm, hbm, hbm, hbm, hbm, hbm],
        out_specs=vmem,
        out_shape=jax.ShapeDtypeStruct((T, Dm), jnp.float32),
        scratch_shapes=[
            pltpu.VMEM((NBUF, HC, Dm), jnp.float32),
            pltpu.VMEM((NBUF, HC, Dm), jnp.float32),
            pltpu.VMEM((NBUF, Dm, HC), jnp.float32),
            pltpu.VMEM((T, Dm), jnp.bfloat16),
            pltpu.VMEM((T, E), jnp.float32),
            pltpu.VMEM((T, Dm), jnp.float32),
            pltpu.SemaphoreType.DMA((NBUF, 3)),
        ],
    )(xf, time_c, ada_w, ada_b, gate_w, w1, w3, w2, sw1, sw3, sw2)

    return y.reshape(B, L, Dm)


# speedup vs baseline: 1.0674x; 1.0169x over previous
import jax
import jax.numpy as jnp
from jax.experimental import pallas as pl
from jax.experimental.pallas import tpu as pltpu

D = 768
FF = 3072
E = 16
HC = 1024
NHALF = FF // HC
NCHUNK = NHALF * E + NHALF
NBUF = 4


def _fused_kernel(xf_ref, time_c_ref, ada_w_ref, ada_b_ref, gate_w_ref,
                  w1h, w3h, w2h, swh1, swh3, swh2,
                  out_ref,
                  b1, b3, b2, h_ref, routing_ref, acc_ref, sems):
    T = xf_ref.shape[0]
    B = time_c_ref.shape[0]
    L = T // B

    def copy_descs(i, slot):
        e = jnp.minimum(i // NHALF, E - 1)
        half = (i % NHALF) * HC
        routed = [
            pltpu.make_async_copy(w1h.at[e, pl.ds(half, HC), :],
                                  b1.at[slot], sems.at[slot, 0]),
            pltpu.make_async_copy(w3h.at[e, pl.ds(half, HC), :],
                                  b3.at[slot], sems.at[slot, 1]),
            pltpu.make_async_copy(w2h.at[e, :, pl.ds(half, HC)],
                                  b2.at[slot], sems.at[slot, 2]),
        ]
        shared = [
            pltpu.make_async_copy(swh1.at[pl.ds(half, HC), :],
                                  b1.at[slot], sems.at[slot, 0]),
            pltpu.make_async_copy(swh3.at[pl.ds(half, HC), :],
                                  b3.at[slot], sems.at[slot, 1]),
            pltpu.make_async_copy(swh2.at[:, pl.ds(half, HC)],
                                  b2.at[slot], sems.at[slot, 2]),
        ]
        return routed, shared

    def start_chunk(i, slot):
        routed, shared = copy_descs(i, slot)

        @pl.when(i < NHALF * E)
        def _():
            for c in routed:
                c.start()

        @pl.when(i >= NHALF * E)
        def _():
            for c in shared:
                c.start()

    def wait_chunk(i, slot):
        routed, shared = copy_descs(i, slot)

        @pl.when(i < NHALF * E)
        def _():
            for c in routed:
                c.wait()

        @pl.when(i >= NHALF * E)
        def _():
            for c in shared:
                c.wait()

    cond = jax.nn.silu(time_c_ref[:])
    ss = jax.lax.dot_general(cond, ada_w_ref[:], (((1,), (1,)), ((), ())),
                             preferred_element_type=jnp.float32)
    ss = ss + ada_b_ref[:][None, :]
    shift = ss[:, :D]
    scale = ss[:, D:]
    rows = jax.lax.broadcasted_iota(jnp.int32, (T, B), 0) // L
    cols = jax.lax.broadcasted_iota(jnp.int32, (T, B), 1)
    P = (rows == cols).astype(jnp.float32)
    shift_t = jax.lax.dot_general(P, shift, (((1,), (0,)), ((), ())),
                                  preferred_element_type=jnp.float32)
    scale_t = jax.lax.dot_general(P, scale, (((1,), (0,)), ((), ())),
                                  preferred_element_type=jnp.float32)
    xf = xf_ref[:]
    m = jnp.mean(xf, axis=-1, keepdims=True)
    v = jnp.mean((xf - m) ** 2, axis=-1, keepdims=True)
    xln = (xf - m) * jax.lax.rsqrt(v + 1e-5)
    h = xln * (1.0 + scale_t) + shift_t
    h_ref[:] = h.astype(jnp.bfloat16)

    logits = jax.lax.dot_general(h, gate_w_ref[:], (((1,), (1,)), ((), ())),
                                 preferred_element_type=jnp.float32)
    ids = jax.lax.broadcasted_iota(jnp.int32, (T, E), 1)
    l1 = jnp.max(logits, axis=1, keepdims=True)
    i1 = jnp.min(jnp.where(logits == l1, ids, E), axis=1, keepdims=True)
    is1 = ids == i1
    masked = jnp.where(is1, jnp.float32(-1e30), logits)
    l2 = jnp.max(masked, axis=1, keepdims=True)
    i2 = jnp.min(jnp.where(masked == l2, ids, E), axis=1, keepdims=True)
    is2 = ids == i2
    wa = jax.nn.sigmoid(l1 - l2)
    routing_ref[:] = jnp.where(is1, wa, 0.0) + jnp.where(is2, 1.0 - wa, 0.0)

    acc_ref[:] = xf

    for i in range(NBUF - 1):
        start_chunk(jnp.int32(i), jnp.int32(i))

    def body(i, carry):
        slot = jax.lax.rem(i, NBUF)
        wait_chunk(i, slot)
        nxt = i + (NBUF - 1)

        @pl.when(nxt < NCHUNK)
        def _():
            start_chunk(nxt, jax.lax.rem(nxt, NBUF))

        hb = h_ref[:]
        g = jax.lax.dot_general(hb, b1[slot].astype(jnp.bfloat16),
                                (((1,), (1,)), ((), ())),
                                preferred_element_type=jnp.float32)
        u = jax.lax.dot_general(hb, b3[slot].astype(jnp.bfloat16),
                                (((1,), (1,)), ((), ())),
                                preferred_element_type=jnp.float32)
        act = (jax.nn.silu(g) * u).astype(jnp.bfloat16)
        partial = jax.lax.dot_general(act, b2[slot].astype(jnp.bfloat16),
                                      (((1,), (1,)), ((), ())),
                                      preferred_element_type=jnp.float32)
        e = i // NHALF
        lane = jax.lax.broadcasted_iota(jnp.int32, (T, E), 1)
        col = jnp.sum(jnp.where(lane == e, routing_ref[:], 0.0), axis=1,
                      keepdims=True)
        col = col + (i >= NHALF * E).astype(jnp.float32)
        acc_ref[:] += partial * col
        return carry

    jax.lax.fori_loop(0, NCHUNK, body, 0)
    out_ref[:] = acc_ref[:]


def kernel(x, time_c, ada_w, ada_b, gate_w, w1, w3, w2, sw1, sw3, sw2):
    B, L, Dm = x.shape
    T = B * L
    xf = x.reshape(T, Dm)

    vmem = pl.BlockSpec(memory_space=pltpu.MemorySpace.VMEM)
    hbm = pl.BlockSpec(memory_space=pltpu.MemorySpace.HBM)

    y = pl.pallas_call(
        _fused_kernel,
        in_specs=[vmem, vmem, vmem, vmem, vmem, hbm, hbm, hbm, hbm, hbm, hbm],
        out_specs=vmem,
        out_shape=jax.ShapeDtypeStruct((T, Dm), jnp.float32),
        scratch_shapes=[
            pltpu.VMEM((NBUF, HC, Dm), jnp.float32),
            pltpu.VMEM((NBUF, HC, Dm), jnp.float32),
            pltpu.VMEM((NBUF, Dm, HC), jnp.float32),
            pltpu.VMEM((T, Dm), jnp.bfloat16),
            pltpu.VMEM((T, E), jnp.float32),
            pltpu.VMEM((T, Dm), jnp.float32),
            pltpu.SemaphoreType.DMA((NBUF, 3)),
        ],
    )(xf, time_c, ada_w, ada_b, gate_w, w1, w3, w2, sw1, sw3, sw2)

    return y.reshape(B, L, Dm)
